# baseline (speedup 1.0000x reference)
import jax
import jax.numpy as jnp
from jax.experimental import pallas as pl
from jax.experimental.pallas import tpu as pltpu

_EPS = 1e-6


def _ln_kernel(gamma_ref, beta_ref, x_ref, o_ref):
    x = x_ref[...].astype(jnp.float32)
    h = x.shape[-1]
    s1 = jnp.sum(x, axis=-1, keepdims=True)
    s2 = jnp.sum(x * x, axis=-1, keepdims=True)
    mean = s1 * (1.0 / h)
    ssq = s2 - s1 * mean
    std = jnp.sqrt(ssq * (1.0 / max(h - 1, 1)))
    scale = gamma_ref[0, 0] * pl.reciprocal(std + _EPS, approx=True)
    o_ref[...] = ((x - mean) * scale + beta_ref[0, 0]).astype(o_ref.dtype)


def _layer_norm(x, gamma, beta, *, tile_rows=512):
    orig_shape = x.shape
    H = orig_shape[-1]
    xf = x.reshape(-1, H)
    R = xf.shape[0]
    dtype = x.dtype

    g = jnp.asarray(gamma, jnp.float32).reshape(1, 1)
    b = jnp.asarray(beta, jnp.float32).reshape(1, 1)

    tile_rows = min(tile_rows, max(8, -(-R // 8) * 8))
    num_tiles = pl.cdiv(R, tile_rows)
    padded_rows = num_tiles * tile_rows
    if padded_rows != R:
        xf = jnp.pad(xf, ((0, padded_rows - R), (0, 0)))

    smem = pl.BlockSpec(memory_space=pltpu.MemorySpace.SMEM)
    out = pl.pallas_call(
        _ln_kernel,
        out_shape=jax.ShapeDtypeStruct((padded_rows, H), dtype),
        grid=(num_tiles,),
        in_specs=[smem, smem, pl.BlockSpec((tile_rows, H), lambda i: (i, 0))],
        out_specs=pl.BlockSpec((tile_rows, H), lambda i: (i, 0)),
        compiler_params=pltpu.CompilerParams(
            dimension_semantics=("parallel",),
            vmem_limit_bytes=64 << 20,
        ),
    )(g, b, xf)

    return out[:R].reshape(orig_shape)


def kernel(x, gamma, beta):
    return _layer_norm(x, gamma, beta)


# speedup vs baseline: 1.1139x; 1.1139x over previous
import jax
import jax.numpy as jnp
from jax.experimental import pallas as pl
from jax.experimental.pallas import tpu as pltpu

_EPS = 1e-6


def _ln_kernel(gamma_ref, beta_ref, x_ref, o_ref):
    x = x_ref[...].astype(jnp.float32)
    h = x.shape[-1]
    s1 = jnp.sum(x, axis=-1, keepdims=True)
    s2 = jnp.sum(x * x, axis=-1, keepdims=True)
    mean = s1 * (1.0 / h)
    ssq = s2 - s1 * mean
    std = jnp.sqrt(ssq * (1.0 / max(h - 1, 1)))
    scale = gamma_ref[0, 0] * pl.reciprocal(std + _EPS, approx=True)
    o_ref[...] = ((x - mean) * scale + beta_ref[0, 0]).astype(o_ref.dtype)


def _layer_norm(x, gamma, beta, *, tile_rows=1024):
    orig_shape = x.shape
    H = orig_shape[-1]
    xf = x.reshape(-1, H)
    R = xf.shape[0]
    dtype = x.dtype

    g = jnp.asarray(gamma, jnp.float32).reshape(1, 1)
    b = jnp.asarray(beta, jnp.float32).reshape(1, 1)

    tile_rows = min(tile_rows, max(8, -(-R // 8) * 8))
    num_tiles = pl.cdiv(R, tile_rows)
    padded_rows = num_tiles * tile_rows
    if padded_rows != R:
        xf = jnp.pad(xf, ((0, padded_rows - R), (0, 0)))

    smem = pl.BlockSpec(memory_space=pltpu.MemorySpace.SMEM)
    out = pl.pallas_call(
        _ln_kernel,
        out_shape=jax.ShapeDtypeStruct((padded_rows, H), dtype),
        grid=(num_tiles,),
        in_specs=[smem, smem, pl.BlockSpec((tile_rows, H), lambda i: (i, 0))],
        out_specs=pl.BlockSpec((tile_rows, H), lambda i: (i, 0)),
        compiler_params=pltpu.CompilerParams(
            dimension_semantics=("parallel",),
            vmem_limit_bytes=64 << 20,
        ),
    )(g, b, xf)

    return out[:R].reshape(orig_shape)


def kernel(x, gamma, beta):
    return _layer_norm(x, gamma, beta)


# speedup vs baseline: 1.1359x; 1.0198x over previous
import jax
import jax.numpy as jnp
from jax.experimental import pallas as pl
from jax.experimental.pallas import tpu as pltpu

_EPS = 1e-6


def _ln_kernel(gamma_ref, beta_ref, x_ref, o_ref):
    x = x_ref[...].astype(jnp.float32)
    h = x.shape[-1]
    s1 = jnp.sum(x, axis=-1, keepdims=True)
    s2 = jnp.sum(x * x, axis=-1, keepdims=True)
    mean = s1 * (1.0 / h)
    ssq = s2 - s1 * mean
    std = jnp.sqrt(ssq * (1.0 / max(h - 1, 1)))
    scale = gamma_ref[0, 0] * pl.reciprocal(std + _EPS, approx=True)
    o_ref[...] = ((x - mean) * scale + beta_ref[0, 0]).astype(o_ref.dtype)


def _layer_norm(x, gamma, beta, *, tile_rows=2048):
    orig_shape = x.shape
    H = orig_shape[-1]
    xf = x.reshape(-1, H)
    R = xf.shape[0]
    dtype = x.dtype

    g = jnp.asarray(gamma, jnp.float32).reshape(1, 1)
    b = jnp.asarray(beta, jnp.float32).reshape(1, 1)

    tile_rows = min(tile_rows, max(8, -(-R // 8) * 8))
    num_tiles = pl.cdiv(R, tile_rows)
    padded_rows = num_tiles * tile_rows
    if padded_rows != R:
        xf = jnp.pad(xf, ((0, padded_rows - R), (0, 0)))

    smem = pl.BlockSpec(memory_space=pltpu.MemorySpace.SMEM)
    out = pl.pallas_call(
        _ln_kernel,
        out_shape=jax.ShapeDtypeStruct((padded_rows, H), dtype),
        grid=(num_tiles,),
        in_specs=[smem, smem, pl.BlockSpec((tile_rows, H), lambda i: (i, 0))],
        out_specs=pl.BlockSpec((tile_rows, H), lambda i: (i, 0)),
        compiler_params=pltpu.CompilerParams(
            dimension_semantics=("parallel",),
            vmem_limit_bytes=64 << 20,
        ),
    )(g, b, xf)

    return out[:R].reshape(orig_shape)


def kernel(x, gamma, beta):
    return _layer_norm(x, gamma, beta)
